# baseline (speedup 1.0000x reference)
import jax
import jax.numpy as jnp
from jax.experimental import pallas as pl

BN = 32768


def _body(a_ref, b_ref, o_ref):
    a = a_ref[...]
    o_ref[...] = jnp.where(a == 1.0, 0.0, a) + b_ref[...]


def kernel(x_1, x_2):
    a = x_1.T
    b = x_2.T
    n = a.shape[1]
    out = pl.pallas_call(
        _body,
        grid=(n // BN,),
        in_specs=[
            pl.BlockSpec((16, BN), lambda i: (0, i)),
            pl.BlockSpec((16, BN), lambda i: (0, i)),
        ],
        out_specs=pl.BlockSpec((16, BN), lambda i: (0, i)),
        out_shape=jax.ShapeDtypeStruct((16, n), jnp.float32),
    )(a, b)
    return out.T


# speedup vs baseline: 1.0178x; 1.0178x over previous
import jax
import jax.numpy as jnp
from jax.experimental import pallas as pl

BN = 65536


def _body(a_ref, b_ref, o_ref):
    a = a_ref[...]
    o_ref[...] = jnp.where(a == 1.0, 0.0, a) + b_ref[...]


def kernel(x_1, x_2):
    a = x_1.T
    b = x_2.T
    n = a.shape[1]
    out = pl.pallas_call(
        _body,
        grid=(n // BN,),
        in_specs=[
            pl.BlockSpec((16, BN), lambda i: (0, i)),
            pl.BlockSpec((16, BN), lambda i: (0, i)),
        ],
        out_specs=pl.BlockSpec((16, BN), lambda i: (0, i)),
        out_shape=jax.ShapeDtypeStruct((16, n), jnp.float32),
    )(a, b)
    return out.T
